# baseline (speedup 1.0000x reference)
import functools

import jax
import jax.numpy as jnp
from jax import lax
from jax.experimental import pallas as pl
from jax.experimental.pallas import tpu as pltpu
from jax.experimental.pallas import tpu_sc as plsc

_EMB_DIM = 128
_NUM_SEG = 26
_NUM_PAIR = _NUM_SEG // 2
_BATCH = 16384
_ROW = _NUM_SEG * _EMB_DIM
_CHUNK_ROWS = 128
_NBUF = 3
_DEPTH = 2


def _permute_sc(pooled_embs):
    info = plsc.get_sparse_core_info()
    num_workers = info.num_cores * info.num_subcores
    rows_per_w = _BATCH // num_workers
    n_rchunks = rows_per_w // _CHUNK_ROWS
    mesh = plsc.VectorSubcoreMesh(core_axis_name="c", subcore_axis_name="s")

    @functools.partial(
        pl.kernel,
        mesh=mesh,
        out_type=jax.ShapeDtypeStruct((_BATCH, _ROW), jnp.float32),
        scratch_types=(
            [pltpu.VMEM((_CHUNK_ROWS, 2 * _EMB_DIM), jnp.float32)] * _NBUF
            + [pltpu.SemaphoreType.DMA] * (2 * _NBUF)
        ),
    )
    def k(in_hbm, out_hbm, *scr):
        bufs = scr[:_NBUF]
        gsems = scr[_NBUF : 2 * _NBUF]
        ssems = scr[2 * _NBUF :]
        wid = lax.axis_index("s") * info.num_cores + lax.axis_index("c")
        row_base = wid * rows_per_w

        steps = [
            (p, c) for p in range(_NUM_PAIR) for c in range(n_rchunks)
        ]
        n_steps = len(steps)

        def gathers(t):
            p, c = steps[t]
            rows = pl.ds(row_base + c * _CHUNK_ROWS, _CHUNK_ROWS)
            hs = []
            for half in range(2):
                src = (_NUM_SEG - 1 - 2 * p - half) * _EMB_DIM
                h = pltpu.make_async_copy(
                    in_hbm.at[rows, pl.ds(src, _EMB_DIM)],
                    bufs[t % _NBUF].at[:, pl.ds(half * _EMB_DIM, _EMB_DIM)],
                    gsems[t % _NBUF],
                )
                h.start()
                hs.append(h)
            return hs

        def scatter(t):
            p, c = steps[t]
            h = pltpu.make_async_copy(
                bufs[t % _NBUF],
                out_hbm.at[
                    pl.ds(row_base + c * _CHUNK_ROWS, _CHUNK_ROWS),
                    pl.ds(2 * p * _EMB_DIM, 2 * _EMB_DIM),
                ],
                ssems[t % _NBUF],
            )
            h.start()
            return h

        g_pend = {}
        s_pend = {}
        for t in range(_DEPTH):
            g_pend[t] = gathers(t)
        for t in range(n_steps):
            for h in g_pend.pop(t):
                h.wait()
            s_pend[t] = scatter(t)
            u = t + _DEPTH
            if u < n_steps:
                if u - _NBUF in s_pend:
                    s_pend.pop(u - _NBUF).wait()
                g_pend[u] = gathers(u)
        for t in sorted(s_pend):
            s_pend.pop(t).wait()

    return k(pooled_embs)


def kernel(pooled_embs):
    return _permute_sc(pooled_embs)


# speedup vs baseline: 1.0501x; 1.0501x over previous
import functools

import jax
import jax.numpy as jnp
from jax import lax
from jax.experimental import pallas as pl
from jax.experimental.pallas import tpu as pltpu
from jax.experimental.pallas import tpu_sc as plsc

_EMB_DIM = 128
_NUM_SEG = 26
_BATCH = 16384
_ROW = _NUM_SEG * _EMB_DIM
_CHUNK_ROWS = 128
_NBUF = 4


def _permute_sc(pooled_embs):
    info = plsc.get_sparse_core_info()
    num_workers = info.num_cores * info.num_subcores
    rows_per_w = _BATCH // num_workers
    n_rchunks = rows_per_w // _CHUNK_ROWS
    assert n_rchunks == _NBUF
    mesh = plsc.VectorSubcoreMesh(core_axis_name="c", subcore_axis_name="s")

    @functools.partial(
        pl.kernel,
        mesh=mesh,
        out_type=jax.ShapeDtypeStruct((_BATCH, _ROW), jnp.float32),
        scratch_types=(
            [pltpu.VMEM((_CHUNK_ROWS, _EMB_DIM), jnp.float32)] * _NBUF
            + [pltpu.SemaphoreType.DMA] * (2 * _NBUF)
        ),
    )
    def k(in_hbm, out_hbm, *scr):
        bufs = scr[:_NBUF]
        gsems = scr[_NBUF : 2 * _NBUF]
        ssems = scr[2 * _NBUF :]
        wid = lax.axis_index("s") * info.num_cores + lax.axis_index("c")
        row_base = wid * rows_per_w

        def gather(j, c, slot):
            src_col = (_NUM_SEG - 1 - j) * _EMB_DIM
            h = pltpu.make_async_copy(
                in_hbm.at[
                    pl.ds(row_base + c * _CHUNK_ROWS, _CHUNK_ROWS),
                    pl.ds(src_col, _EMB_DIM),
                ],
                bufs[slot],
                gsems[slot],
            )
            h.start()
            return h

        def scatter(j, c, slot):
            h = pltpu.make_async_copy(
                bufs[slot],
                out_hbm.at[
                    pl.ds(row_base + c * _CHUNK_ROWS, _CHUNK_ROWS),
                    pl.ds(j * _EMB_DIM, _EMB_DIM),
                ],
                ssems[slot],
            )
            h.start()
            return h

        dummy_in = in_hbm.at[pl.ds(0, _CHUNK_ROWS), pl.ds(0, _EMB_DIM)]
        dummy_out = out_hbm.at[pl.ds(0, _CHUNK_ROWS), pl.ds(0, _EMB_DIM)]

        def wait_gather(slot):
            pltpu.make_async_copy(dummy_in, bufs[slot], gsems[slot]).wait()

        def wait_scatter(slot):
            pltpu.make_async_copy(bufs[slot], dummy_out, ssems[slot]).wait()

        gather(0, 0, 0)
        gather(0, 1, 1)
        gather(0, 2, 2)
        wait_gather(0)
        scatter(0, 0, 0)
        gather(0, 3, 3)
        wait_gather(1)
        scatter(0, 1, 1)
        wait_scatter(0)
        gather(1, 0, 0)
        wait_gather(2)
        scatter(0, 2, 2)
        wait_scatter(1)
        gather(1, 1, 1)

        def body(kk, carry):
            for b in range(4):
                if b == 0:
                    j, c = kk, 3
                else:
                    j, c = kk + 1, b - 1
                slot = (3 + b) % 4
                wait_gather(slot)
                scatter(j, c, slot)
                wslot = (2 + b) % 4
                wait_scatter(wslot)
                if b < 2:
                    gather(kk + 1, 2 + b, wslot)
                else:
                    gather(kk + 2, b - 2, wslot)
            return carry

        lax.fori_loop(0, _NUM_SEG - 2, body, 0)

        wait_gather(3)
        scatter(24, 3, 3)
        wait_scatter(2)
        gather(25, 2, 2)
        wait_gather(0)
        scatter(25, 0, 0)
        wait_scatter(3)
        gather(25, 3, 3)
        wait_gather(1)
        scatter(25, 1, 1)
        wait_gather(2)
        scatter(25, 2, 2)
        wait_gather(3)
        scatter(25, 3, 3)
        for slot in range(4):
            wait_scatter(slot)

    return k(pooled_embs)


def kernel(pooled_embs):
    return _permute_sc(pooled_embs)
